# baseline (speedup 1.0000x reference)
import functools

import jax
import jax.numpy as jnp
from jax import lax
from jax.experimental import pallas as pl
from jax.experimental.pallas import tpu as pltpu
from jax.experimental.pallas import tpu_sc as plsc

B, C, H, W = 4, 32, 512, 512
HW = H * W
N = 100000
NQ = B * N
NWK = 32
SB = 128
NBLK = NQ // SB
FULL = NBLK // NWK
EXTRA = NBLK - FULL * NWK
ROUNDS = FULL + 1


def _sc_body(table, qxy, out, qb, idx, wv, rows, out_v, out_vT,
             gsem0, gsem1, osem0, osem1):
    gsem = (gsem0, gsem1)
    osem = (osem0, osem1)
    cid = lax.axis_index("c")
    sid = lax.axis_index("s")
    wid = sid * 2 + cid
    start = wid * FULL + jnp.minimum(wid, EXTRA)
    cnt = jnp.where(wid < EXTRA, FULL + 1, FULL)

    pltpu.sync_copy(qxy.at[pl.ds(start, FULL)], qb.at[pl.ds(0, FULL)])

    @pl.when(cnt == FULL + 1)
    def _():
        pltpu.sync_copy(qxy.at[pl.ds(start + FULL, 1)],
                        qb.at[pl.ds(FULL, 1)])

    def fire(t, s):
        @pl.when(t < cnt)
        def _():
            blk = start + t
            lane = lax.iota(jnp.int32, 16)
            for g in range(SB // 16):
                gx = qb[t, pl.ds(g * 16, 16)]
                gy = qb[t, pl.ds(SB + g * 16, 16)]
                ix = ((gx + 1.0) * W - 1.0) * 0.5
                iy = ((gy + 1.0) * H - 1.0) * 0.5
                ix0 = (ix + 1.0).astype(jnp.int32) - 1
                iy0 = (iy + 1.0).astype(jnp.int32) - 1
                wx1 = ix - ix0.astype(jnp.float32)
                wy1 = iy - iy0.astype(jnp.float32)
                wx0 = 1.0 - wx1
                wy0 = 1.0 - wy1
                ix1 = ix0 + 1
                iy1 = iy0 + 1
                zero = gx * 0.0
                wx0 = jnp.where(ix0 >= 0, wx0, zero)
                wx1 = jnp.where(ix1 <= W - 1, wx1, zero)
                wy0 = jnp.where(iy0 >= 0, wy0, zero)
                wy1 = jnp.where(iy1 <= H - 1, wy1, zero)
                cx0 = jnp.maximum(ix0, 0)
                cx1 = jnp.minimum(ix1, W - 1)
                cy0 = jnp.maximum(iy0, 0)
                cy1 = jnp.minimum(iy1, H - 1)
                gq = blk * SB + g * 16 + lane
                tb = (gq // N) * HW
                gsl = pl.ds(g * 16, 16)
                r0 = tb + cy0 * W
                r1 = tb + cy1 * W
                idx[s, 0, gsl] = r0 + cx0
                idx[s, 1, gsl] = r0 + cx1
                idx[s, 2, gsl] = r1 + cx0
                idx[s, 3, gsl] = r1 + cx1
                wv[s, 0, gsl] = wy0 * wx0
                wv[s, 1, gsl] = wy0 * wx1
                wv[s, 2, gsl] = wy1 * wx0
                wv[s, 3, gsl] = wy1 * wx1
            for c in range(4):
                pltpu.async_copy(table.at[idx.at[s, c]],
                                 rows.at[pl.ds((s * 4 + c) * SB, SB)],
                                 gsem[s])

    def wait_acc_store(t, s):
        @pl.when(t < cnt)
        def _():
            blk = start + t
            for c in range(4):
                pltpu.make_async_copy(
                    table.at[idx.at[s, c]],
                    rows.at[pl.ds((s * 4 + c) * SB, SB)],
                    gsem[s]).wait()

            @pl.when(t >= 2)
            def _():
                pltpu.make_async_copy(out_vT.at[s], out.at[:, pl.ds(0, SB)],
                                      osem[s]).wait()

            @pl.loop(0, SB // 16)
            def acc_group(g):
                w0 = wv[s, 0, pl.ds(g * 16, 16)]
                w1 = wv[s, 1, pl.ds(g * 16, 16)]
                w2 = wv[s, 2, pl.ds(g * 16, 16)]
                w3 = wv[s, 3, pl.ds(g * 16, 16)]
                for q in range(16):
                    qq = g * 16 + q
                    for h in range(C // 16):
                        hsl = pl.ds(h * 16, 16)
                        out_v[s, qq, hsl] = (
                            rows[(s * 4 + 0) * SB + qq, hsl] * w0[q]
                            + rows[(s * 4 + 1) * SB + qq, hsl] * w1[q]
                            + rows[(s * 4 + 2) * SB + qq, hsl] * w2[q]
                            + rows[(s * 4 + 3) * SB + qq, hsl] * w3[q])

            @pl.loop(0, C)
            def tpose(ch):
                chv = jnp.full((16,), ch, jnp.int32)
                lane3 = lax.iota(jnp.int32, 16)
                for qg in range(SB // 16):
                    qvec = qg * 16 + lane3
                    v = plsc.load_gather(out_v.at[s], [qvec, chv])
                    out_vT[s, ch, pl.ds(qg * 16, 16)] = v

            pltpu.async_copy(out_vT.at[s], out.at[:, pl.ds(blk * SB, SB)],
                             osem[s])

    fire(0, 0)

    @pl.loop(0, ROUNDS, step=2)
    def steady(t):
        fire(t + 1, 1)
        wait_acc_store(t, 0)
        fire(t + 2, 0)
        wait_acc_store(t + 1, 1)

    for s in range(2):
        pltpu.make_async_copy(out_vT.at[s], out.at[:, pl.ds(0, SB)],
                              osem[s]).wait()


@jax.jit
def kernel(x, query_pos):
    table = x.transpose(0, 2, 3, 1).reshape(B * HW, C)
    gx = query_pos[..., 1].reshape(NBLK, SB)
    gy = query_pos[..., 0].reshape(NBLK, SB)
    qxy = jnp.concatenate([gx, gy], axis=1)

    mesh = plsc.VectorSubcoreMesh(core_axis_name="c", subcore_axis_name="s")
    run = functools.partial(
        pl.kernel,
        mesh=mesh,
        out_type=jax.ShapeDtypeStruct((C, NQ), jnp.float32),
        compiler_params=pltpu.CompilerParams(
            use_tc_tiling_on_sc=False, needs_layout_passes=False),
        scratch_types=[
            pltpu.VMEM((ROUNDS, 2 * SB), jnp.float32),
            pltpu.VMEM((2, 4, SB), jnp.int32),
            pltpu.VMEM((2, 4, SB), jnp.float32),
            pltpu.VMEM((2 * 4 * SB, C), jnp.float32),
            pltpu.VMEM((2, SB, C + 1), jnp.float32),
            pltpu.VMEM((2, C, SB), jnp.float32),
            pltpu.SemaphoreType.DMA,
            pltpu.SemaphoreType.DMA,
            pltpu.SemaphoreType.DMA,
            pltpu.SemaphoreType.DMA,
        ],
    )(_sc_body)
    return run(table, qxy).T


# speedup vs baseline: 1.3097x; 1.3097x over previous
import functools

import jax
import jax.numpy as jnp
from jax import lax
from jax.experimental import pallas as pl
from jax.experimental.pallas import tpu as pltpu
from jax.experimental.pallas import tpu_sc as plsc

B, C, H, W = 4, 32, 512, 512
HW = H * W
N = 100000
NQ = B * N
NWK = 32
SB = 128
NBLK = NQ // SB
FULL = NBLK // NWK
EXTRA = NBLK - FULL * NWK
ROUNDS = FULL + 1

NTY = H // 8
NTX = W // 128
NST = B * NTY * NTX
STPW = NST // NWK
HTPW = 2 * STPW


def _tp_body(z, table, zt, tbuf, zsem0, zsem1, osem0, osem1):
    zsem = (zsem0, zsem1)
    osem = (osem0, osem1)
    cid = lax.axis_index("c")
    sid = lax.axis_index("s")
    wid = sid * 2 + cid
    lane = lax.iota(jnp.int32, 16)

    def decode(ht):
        stl = ht >> 1
        st = wid * STPW + stl
        b = st // (NTY * NTX)
        rem = st % (NTY * NTX)
        yt = rem // NTX
        xt = rem % NTX
        return stl, b, yt, xt

    def fire(ht, p, half):
        @pl.when(ht < HTPW)
        def _():
            stl, b, yt, xt = decode(ht)
            zbase = ((b * C + half * 16) * NTY + yt) * NTX + xt
            for i in range(16):
                pltpu.async_copy(z.at[zbase + i * (NTY * NTX)],
                                 zt.at[p, i, pl.ds(0, 1024)], zsem[p])

    def proc(ht, p, half, sp):
        @pl.when(ht < HTPW)
        def _():
            stl, b, yt, xt = decode(ht)
            for i in range(16):
                pltpu.make_async_copy(z.at[0], zt.at[p, i, pl.ds(0, 1024)],
                                      zsem[p]).wait()

            if half == 0:
                @pl.when(stl >= 2)
                def _():
                    for yr in range(8):
                        pltpu.make_async_copy(tbuf.at[sp, pl.ds(0, SB)],
                                              table.at[pl.ds(0, SB)],
                                              osem[sp]).wait()

            @pl.loop(0, 64)
            def pgrp(pg):
                for k in range(16):
                    p_ = pg * 16 + k
                    v = plsc.load_gather(
                        zt.at[p], [lane, jnp.full((16,), p_, jnp.int32)])
                    tbuf[sp, p_, pl.ds(half * 16, 16)] = v

            if half == 1:
                y0 = b * H + yt * 8
                for yr in range(8):
                    pltpu.async_copy(
                        tbuf.at[sp, pl.ds(yr * SB, SB)],
                        table.at[pl.ds((y0 + yr) * W + xt * SB, SB)],
                        osem[sp])

    fire(0, 0, 0)

    @pl.loop(0, HTPW, step=4)
    def steady(t):
        fire(t + 1, 1, 1)
        proc(t, 0, 0, 0)
        fire(t + 2, 0, 0)
        proc(t + 1, 1, 1, 0)
        fire(t + 3, 1, 1)
        proc(t + 2, 0, 0, 1)
        fire(t + 4, 0, 0)
        proc(t + 3, 1, 1, 1)

    for sp in range(2):
        for yr in range(8):
            pltpu.make_async_copy(tbuf.at[sp, pl.ds(0, SB)],
                                  table.at[pl.ds(0, SB)], osem[sp]).wait()


def _sc_body(table, qxy, out, qb, idx, wv, rows, out_vT,
             gsem0, gsem1, osem0, osem1):
    gsem = (gsem0, gsem1)
    osem = (osem0, osem1)
    cid = lax.axis_index("c")
    sid = lax.axis_index("s")
    wid = sid * 2 + cid
    start = wid * FULL + jnp.minimum(wid, EXTRA)
    cnt = jnp.where(wid < EXTRA, FULL + 1, FULL)

    pltpu.sync_copy(qxy.at[pl.ds(start, FULL)], qb.at[pl.ds(0, FULL)])

    @pl.when(cnt == FULL + 1)
    def _():
        pltpu.sync_copy(qxy.at[pl.ds(start + FULL, 1)],
                        qb.at[pl.ds(FULL, 1)])

    def fire(t, s):
        @pl.when(t < cnt)
        def _():
            blk = start + t
            lane = lax.iota(jnp.int32, 16)
            for g in range(SB // 16):
                gx = qb[t, pl.ds(g * 16, 16)]
                gy = qb[t, pl.ds(SB + g * 16, 16)]
                ix = ((gx + 1.0) * W - 1.0) * 0.5
                iy = ((gy + 1.0) * H - 1.0) * 0.5
                ix0 = (ix + 1.0).astype(jnp.int32) - 1
                iy0 = (iy + 1.0).astype(jnp.int32) - 1
                wx1 = ix - ix0.astype(jnp.float32)
                wy1 = iy - iy0.astype(jnp.float32)
                wx0 = 1.0 - wx1
                wy0 = 1.0 - wy1
                ix1 = ix0 + 1
                iy1 = iy0 + 1
                zero = gx * 0.0
                wx0 = jnp.where(ix0 >= 0, wx0, zero)
                wx1 = jnp.where(ix1 <= W - 1, wx1, zero)
                wy0 = jnp.where(iy0 >= 0, wy0, zero)
                wy1 = jnp.where(iy1 <= H - 1, wy1, zero)
                cx0 = jnp.maximum(ix0, 0)
                cx1 = jnp.minimum(ix1, W - 1)
                cy0 = jnp.maximum(iy0, 0)
                cy1 = jnp.minimum(iy1, H - 1)
                gq = blk * SB + g * 16 + lane
                tb = (gq // N) * HW
                gsl = pl.ds(g * 16, 16)
                r0 = tb + cy0 * W
                r1 = tb + cy1 * W
                idx[s, 0, gsl] = r0 + cx0
                idx[s, 1, gsl] = r0 + cx1
                idx[s, 2, gsl] = r1 + cx0
                idx[s, 3, gsl] = r1 + cx1
                wv[s, 0, gsl] = wy0 * wx0
                wv[s, 1, gsl] = wy0 * wx1
                wv[s, 2, gsl] = wy1 * wx0
                wv[s, 3, gsl] = wy1 * wx1
            for c in range(4):
                pltpu.async_copy(table.at[idx.at[s, c]],
                                 rows.at[pl.ds((s * 4 + c) * SB, SB)],
                                 gsem[s])

    def wait_acc_store(t, s):
        @pl.when(t < cnt)
        def _():
            blk = start + t
            for c in range(4):
                pltpu.make_async_copy(
                    table.at[idx.at[s, c]],
                    rows.at[pl.ds((s * 4 + c) * SB, SB)],
                    gsem[s]).wait()

            @pl.when(t >= 2)
            def _():
                pltpu.make_async_copy(out_vT.at[s, :, pl.ds(0, SB)],
                                      out.at[:, pl.ds(0, SB)],
                                      osem[s]).wait()

            @pl.loop(0, SB // 16)
            def acc_group(g):
                lane2 = lax.iota(jnp.int32, 16)
                w0 = wv[s, 0, pl.ds(g * 16, 16)]
                w1 = wv[s, 1, pl.ds(g * 16, 16)]
                w2 = wv[s, 2, pl.ds(g * 16, 16)]
                w3 = wv[s, 3, pl.ds(g * 16, 16)]
                for q in range(16):
                    qq = g * 16 + q
                    for h in range(C // 16):
                        hsl = pl.ds(h * 16, 16)
                        v = (rows[(s * 4 + 0) * SB + qq, hsl] * w0[q]
                             + rows[(s * 4 + 1) * SB + qq, hsl] * w1[q]
                             + rows[(s * 4 + 2) * SB + qq, hsl] * w2[q]
                             + rows[(s * 4 + 3) * SB + qq, hsl] * w3[q])
                        plsc.store_scatter(
                            out_vT.at[s],
                            [h * 16 + lane2, jnp.full((16,), qq, jnp.int32)],
                            v)

            pltpu.async_copy(out_vT.at[s, :, pl.ds(0, SB)],
                             out.at[:, pl.ds(blk * SB, SB)],
                             osem[s])

    fire(0, 0)

    @pl.loop(0, ROUNDS, step=2)
    def steady(t):
        fire(t + 1, 1)
        wait_acc_store(t, 0)
        fire(t + 2, 0)
        wait_acc_store(t + 1, 1)

    for s in range(2):
        pltpu.make_async_copy(out_vT.at[s, :, pl.ds(0, SB)],
                              out.at[:, pl.ds(0, SB)],
                              osem[s]).wait()


@jax.jit
def kernel(x, query_pos):
    zview = (x.reshape(B, C, NTY, 8, NTX, 128)
             .transpose(0, 1, 2, 4, 3, 5)
             .reshape(B * C * NTY * NTX, 1024))
    gx = query_pos[..., 1].reshape(NBLK, SB)
    gy = query_pos[..., 0].reshape(NBLK, SB)
    qxy = jnp.concatenate([gx, gy], axis=1)

    mesh = plsc.VectorSubcoreMesh(core_axis_name="c", subcore_axis_name="s")
    tp_run = functools.partial(
        pl.kernel,
        mesh=mesh,
        out_type=jax.ShapeDtypeStruct((B * HW, C), jnp.float32),
        compiler_params=pltpu.CompilerParams(
            use_tc_tiling_on_sc=False, needs_layout_passes=False),
        scratch_types=[
            pltpu.VMEM((2, 16, 1025), jnp.float32),
            pltpu.VMEM((2, 8 * SB, C), jnp.float32),
            pltpu.SemaphoreType.DMA,
            pltpu.SemaphoreType.DMA,
            pltpu.SemaphoreType.DMA,
            pltpu.SemaphoreType.DMA,
        ],
    )(_tp_body)
    table = tp_run(zview)

    run = functools.partial(
        pl.kernel,
        mesh=mesh,
        out_type=jax.ShapeDtypeStruct((C, NQ), jnp.float32),
        compiler_params=pltpu.CompilerParams(
            use_tc_tiling_on_sc=False, needs_layout_passes=False),
        scratch_types=[
            pltpu.VMEM((ROUNDS, 2 * SB), jnp.float32),
            pltpu.VMEM((2, 4, SB), jnp.int32),
            pltpu.VMEM((2, 4, SB), jnp.float32),
            pltpu.VMEM((2 * 4 * SB, C), jnp.float32),
            pltpu.VMEM((2, C, SB + 1), jnp.float32),
            pltpu.SemaphoreType.DMA,
            pltpu.SemaphoreType.DMA,
            pltpu.SemaphoreType.DMA,
            pltpu.SemaphoreType.DMA,
        ],
    )(_sc_body)
    return run(table, qxy).T
